# baseline (speedup 1.0000x reference)
import functools

import jax
import jax.numpy as jnp
from jax import lax
from jax.experimental import pallas as pl
from jax.experimental.pallas import tpu as pltpu
from jax.experimental.pallas import tpu_sc as plsc

VOCAB = 30522
HIDDEN = 768
MAX_POS = 2048
B, S = 4, 2048
EPS = 1e-12

NUM_TOKENS = B * S
NC, NS = 2, 16
NW = NC * NS
NSLICE = 4
S_SLC = S // NSLICE
TOK_SLC = B * S_SLC
TOK_PER_W = TOK_SLC // NW

_sc_mesh = plsc.VectorSubcoreMesh(core_axis_name="c", subcore_axis_name="s")


@functools.partial(
    pl.kernel,
    out_type=jax.ShapeDtypeStruct((TOK_SLC, HIDDEN), jnp.float32),
    mesh=_sc_mesh,
    scratch_types=[
        pltpu.VMEM((TOK_PER_W,), jnp.int32),
        pltpu.VMEM((TOK_PER_W, HIDDEN), jnp.float32),
        pltpu.SemaphoreType.DMA,
    ],
)
def _sc_gather(table_hbm, idx_hbm, out_hbm, idx_v, rows_v, sem):
    wid = lax.axis_index("s") * NC + lax.axis_index("c")
    base = wid * TOK_PER_W
    pltpu.sync_copy(idx_hbm.at[pl.ds(base, TOK_PER_W)], idx_v)
    pltpu.async_copy(table_hbm.at[idx_v], rows_v, sem).wait()
    pltpu.sync_copy(rows_v, out_hbm.at[pl.ds(base, TOK_PER_W)])


ROWS_BLK = 512


def _ln_body(words_ref, pos_ref, tt_ref, type_ref, gamma_ref, beta_ref, out_ref):
    t0 = type_ref[0:1, :]
    tdiff = type_ref[1:2, :] - t0
    ttf = tt_ref[...].astype(jnp.float32)
    x = words_ref[...] + pos_ref[...] + t0 + ttf * tdiff
    mean = jnp.mean(x, axis=-1, keepdims=True)
    xc = x - mean
    var = jnp.mean(xc * xc, axis=-1, keepdims=True)
    normed = xc * lax.rsqrt(var + EPS)
    out_ref[...] = normed * gamma_ref[...] + beta_ref[...]


def _ln_body_acc(acc_ref, words_ref, pos_ref, tt_ref, type_ref, gamma_ref,
                 beta_ref, out_ref):
    del acc_ref
    _ln_body(words_ref, pos_ref, tt_ref, type_ref, gamma_ref, beta_ref, out_ref)


def _make_ln_call(k, aliased):
    specs = [
        pl.BlockSpec((ROWS_BLK, HIDDEN), lambda i: (i, 0)),
        pl.BlockSpec((S_SLC, HIDDEN), lambda i: (0, 0)),
        pl.BlockSpec((ROWS_BLK, 1), lambda i: (i, 0)),
        pl.BlockSpec((2, HIDDEN), lambda i: (0, 0)),
        pl.BlockSpec((1, HIDDEN), lambda i: (0, 0)),
        pl.BlockSpec((1, HIDDEN), lambda i: (0, 0)),
    ]
    out_spec = pl.BlockSpec((ROWS_BLK, HIDDEN), lambda i: (i * NSLICE + k, 0))
    if aliased:
        return pl.pallas_call(
            _ln_body_acc,
            grid=(B,),
            in_specs=[pl.BlockSpec(memory_space=pl.ANY)] + specs,
            out_specs=out_spec,
            out_shape=jax.ShapeDtypeStruct((NUM_TOKENS, HIDDEN), jnp.float32),
            input_output_aliases={0: 0},
        )
    return pl.pallas_call(
        _ln_body,
        grid=(B,),
        in_specs=specs,
        out_specs=out_spec,
        out_shape=jax.ShapeDtypeStruct((NUM_TOKENS, HIDDEN), jnp.float32),
    )


_ln_first = _make_ln_call(0, aliased=False)
_ln_rest = [_make_ln_call(k, aliased=True) for k in range(1, NSLICE)]


@jax.jit
def kernel(input_ids, token_type_ids, word_emb, pos_emb, type_emb, gamma, beta):
    ids32 = input_ids.astype(jnp.int32)
    tt32 = token_type_ids.astype(jnp.int32)
    gamma2 = gamma.reshape(1, HIDDEN)
    beta2 = beta.reshape(1, HIDDEN)

    words = []
    tts = []
    for k in range(NSLICE):
        idx_k = ids32[:, k * S_SLC:(k + 1) * S_SLC].reshape(TOK_SLC)
        words.append(_sc_gather(word_emb, idx_k))
        tts.append(tt32[:, k * S_SLC:(k + 1) * S_SLC].reshape(TOK_SLC, 1))

    pos_slices = [pos_emb[k * S_SLC:(k + 1) * S_SLC] for k in range(NSLICE)]

    acc = _ln_first(words[0], pos_slices[0], tts[0], type_emb, gamma2, beta2)
    for k in range(1, NSLICE):
        acc = _ln_rest[k - 1](acc, words[k], pos_slices[k], tts[k], type_emb,
                              gamma2, beta2)
    return acc.reshape(B, S, HIDDEN)


# speedup vs baseline: 1.1373x; 1.1373x over previous
import functools

import jax
import jax.numpy as jnp
from jax import lax
from jax.experimental import pallas as pl
from jax.experimental.pallas import tpu as pltpu
from jax.experimental.pallas import tpu_sc as plsc

VOCAB = 30522
HIDDEN = 768
MAX_POS = 2048
B, S = 4, 2048
EPS = 1e-12

NUM_TOKENS = B * S
NC, NS = 2, 16
NW = NC * NS
TOK_PER_W = NUM_TOKENS // NW
SEG_PER_B = S // TOK_PER_W
CHUNK = 64
NCHUNK = TOK_PER_W // CHUNK

_sc_mesh = plsc.VectorSubcoreMesh(core_axis_name="c", subcore_axis_name="s")


@functools.partial(
    pl.kernel,
    out_type=jax.ShapeDtypeStruct((NUM_TOKENS, HIDDEN), jnp.float32),
    mesh=_sc_mesh,
    scratch_types=[
        pltpu.VMEM((TOK_PER_W,), jnp.int32),
        pltpu.VMEM((CHUNK, HIDDEN), jnp.float32),
        pltpu.VMEM((CHUNK, HIDDEN), jnp.float32),
        pltpu.SemaphoreType.DMA,
        pltpu.SemaphoreType.DMA,
        pltpu.SemaphoreType.DMA,
        pltpu.SemaphoreType.DMA,
    ],
)
def _sc_gather(table_hbm, ids_hbm, out_hbm, idx_v, buf0, buf1, gs0, gs1, ws0, ws1):
    wid = lax.axis_index("s") * NC + lax.axis_index("c")
    b = wid // SEG_PER_B
    s0 = (wid % SEG_PER_B) * TOK_PER_W
    base = wid * TOK_PER_W
    pltpu.sync_copy(ids_hbm.at[b, pl.ds(s0, TOK_PER_W)], idx_v)

    bufs = (buf0, buf1)
    gsems = (gs0, gs1)
    wsems = (ws0, ws1)

    def gather_start(c):
        return pltpu.async_copy(
            table_hbm.at[idx_v.at[pl.ds(c * CHUNK, CHUNK)]], bufs[c % 2],
            gsems[c % 2])

    def write_start(c):
        return pltpu.async_copy(
            bufs[c % 2], out_hbm.at[pl.ds(base + c * CHUNK, CHUNK)],
            wsems[c % 2])

    g = [gather_start(0), gather_start(1)]
    w = []
    for c in range(NCHUNK):
        g[c].wait()
        w.append(write_start(c))
        if c + 2 < NCHUNK:
            w[c].wait()
            g.append(gather_start(c + 2))
    w[-2].wait()
    w[-1].wait()


ROWS_BLK = 512
S_BLKS = S // ROWS_BLK


def _ln_body(words_ref, pos_ref, tt_ref, type_ref, gamma_ref, beta_ref, out_ref):
    t0 = type_ref[0:1, :]
    tdiff = type_ref[1:2, :] - t0
    ttf = tt_ref[...].astype(jnp.float32).reshape(ROWS_BLK, 1)
    x = words_ref[...] + pos_ref[...] + t0 + ttf * tdiff
    mean = jnp.mean(x, axis=-1, keepdims=True)
    xc = x - mean
    var = jnp.mean(xc * xc, axis=-1, keepdims=True)
    normed = xc * lax.rsqrt(var + EPS)
    out_ref[...] = normed * gamma_ref[...] + beta_ref[...]


_ln_call = pl.pallas_call(
    _ln_body,
    grid=(S_BLKS, B),
    in_specs=[
        pl.BlockSpec((ROWS_BLK, HIDDEN), lambda i, j: (j * S_BLKS + i, 0)),
        pl.BlockSpec((ROWS_BLK, HIDDEN), lambda i, j: (i, 0)),
        pl.BlockSpec((1, 1, ROWS_BLK), lambda i, j: (j, 0, i)),
        pl.BlockSpec((2, HIDDEN), lambda i, j: (0, 0)),
        pl.BlockSpec((1, HIDDEN), lambda i, j: (0, 0)),
        pl.BlockSpec((1, HIDDEN), lambda i, j: (0, 0)),
    ],
    out_specs=pl.BlockSpec((ROWS_BLK, HIDDEN), lambda i, j: (j * S_BLKS + i, 0)),
    out_shape=jax.ShapeDtypeStruct((NUM_TOKENS, HIDDEN), jnp.float32),
)


@jax.jit
def kernel(input_ids, token_type_ids, word_emb, pos_emb, type_emb, gamma, beta):
    ids32 = input_ids.astype(jnp.int32)
    tt32 = token_type_ids.astype(jnp.int32).reshape(B, 1, S)
    words = _sc_gather(word_emb, ids32)
    out = _ln_call(
        words,
        pos_emb,
        tt32,
        type_emb,
        gamma.reshape(1, HIDDEN),
        beta.reshape(1, HIDDEN),
    )
    return out.reshape(B, S, HIDDEN)
